# Initial kernel scaffold; baseline (speedup 1.0000x reference)
import functools

import jax
import jax.numpy as jnp
from jax import lax
from jax.experimental import pallas as pl
from jax.experimental.pallas import tpu as pltpu
from jax.experimental.pallas import tpu_sc as plsc

N = 10000
E = 320000
H = 128
OUT = 64
G = 64

NC = 2
NS = 16
NW = NC * NS
C = 128
CHUNKS_W = -(-E // (C * NW))
E_PAD = CHUNKS_W * C * NW
N_PAD = 10240
ROWS_S = N_PAD // NS

_mesh = plsc.VectorSubcoreMesh(core_axis_name="c", subcore_axis_name="s")


def _worker_id():
    return lax.axis_index("s") * NC + lax.axis_index("c")


@functools.partial(
    pl.kernel,
    mesh=_mesh,
    out_type=(
        jax.ShapeDtypeStruct((E_PAD,), jnp.float32),
        jax.ShapeDtypeStruct((NC, N_PAD), jnp.float32),
    ),
    scratch_types=[
        pltpu.VMEM((C,), jnp.int32),
        pltpu.VMEM((C,), jnp.int32),
        pltpu.VMEM((C,), jnp.float32),
        pltpu.VMEM((C, H), jnp.float32),
        pltpu.VMEM((C, H), jnp.float32),
        pltpu.VMEM((C,), jnp.float32),
        pltpu.VMEM((H,), jnp.float32),
        pltpu.VMEM((H,), jnp.float32),
        pltpu.VMEM_SHARED((N_PAD,), jnp.float32),
        pltpu.SemaphoreType.DMA,
        pltpu.SemaphoreType.DMA,
    ],
)
def _sc_pass_a(hl_hbm, hr_hbm, src_hbm, dst_hbm, ea_hbm, we_hbm, att_hbm,
               zeros1_hbm, ex_hbm, den_hbm,
               src_v, dst_v, ea_v, rl_v, rr_v, ex_v, we_v, att_v,
               den_sh, sem1, sem2):
    c = lax.axis_index("c")
    s = lax.axis_index("s")
    wid = _worker_id()

    pltpu.sync_copy(we_hbm, we_v)
    pltpu.sync_copy(att_hbm, att_v)
    pltpu.sync_copy(zeros1_hbm.at[pl.ds(s * ROWS_S, ROWS_S)],
                    den_sh.at[pl.ds(s * ROWS_S, ROWS_S)])
    plsc.subcore_barrier()

    def chunk_body(j, carry):
        base = (j * NW + wid) * C
        pltpu.sync_copy(src_hbm.at[pl.ds(base, C)], src_v)
        pltpu.sync_copy(dst_hbm.at[pl.ds(base, C)], dst_v)
        pltpu.sync_copy(ea_hbm.at[pl.ds(base, C)], ea_v)
        cp1 = pltpu.async_copy(hl_hbm.at[src_v], rl_v, sem1)
        cp2 = pltpu.async_copy(hr_hbm.at[dst_v], rr_v, sem2)
        cp1.wait()
        cp2.wait()
        lanes = lax.iota(jnp.int32, 16)
        for g in range(C // 16):
            rowi = lanes + (g * 16)
            ea16 = ea_v[pl.ds(g * 16, 16)]
            eid16 = lanes + (base + g * 16)

            def f_body(f, acc):
                fv = jnp.full((16,), f, dtype=jnp.int32)
                lv = plsc.load_gather(rl_v, [rowi, fv])
                rv = plsc.load_gather(rr_v, [rowi, fv])
                wf = plsc.load_gather(we_v, [fv])
                af = plsc.load_gather(att_v, [fv])
                m = lv + rv + ea16 * wf
                m = jnp.maximum(m, 0.2 * m)
                return acc + m * af

            acc = lax.fori_loop(0, H, f_body, jnp.zeros((16,), jnp.float32),
                                unroll=8)
            exv = jnp.exp(acc)
            exv = jnp.where(eid16 < E, exv, 0.0)
            ex_v[pl.ds(g * 16, 16)] = exv
        pltpu.sync_copy(ex_v, ex_hbm.at[pl.ds(base, C)])
        pltpu.sync_copy(ex_v, den_sh.at[dst_v], add=True)
        return carry

    lax.fori_loop(0, CHUNKS_W, chunk_body, 0)
    plsc.subcore_barrier()
    pltpu.sync_copy(den_sh.at[pl.ds(s * ROWS_S, ROWS_S)],
                    den_hbm.at[c, pl.ds(s * ROWS_S, ROWS_S)])


@functools.partial(
    pl.kernel,
    mesh=_mesh,
    out_type=jax.ShapeDtypeStruct((NC, N_PAD, H), jnp.float32),
    scratch_types=[
        pltpu.VMEM((C,), jnp.int32),
        pltpu.VMEM((C,), jnp.int32),
        pltpu.VMEM((C,), jnp.float32),
        pltpu.VMEM((C,), jnp.float32),
        pltpu.VMEM((C,), jnp.float32),
        pltpu.VMEM((C, H), jnp.float32),
        pltpu.VMEM((ROWS_S,), jnp.float32),
        pltpu.VMEM((ROWS_S,), jnp.float32),
        pltpu.VMEM((ROWS_S,), jnp.float32),
        pltpu.VMEM_SHARED((N_PAD,), jnp.float32),
        pltpu.VMEM_SHARED((N_PAD, H), jnp.float32),
        pltpu.SemaphoreType.DMA,
        pltpu.SemaphoreType.DMA,
    ],
)
def _sc_pass_b(hl_hbm, src_hbm, dst_hbm, ex_hbm, denp_hbm, zeros2_hbm,
               out_hbm,
               src_v, dst_v, ex_v, den_v, al_v, rl_v, d0_v, d1_v, dt_v,
               den_sh, out_sh, sem1, sem2):
    c = lax.axis_index("c")
    s = lax.axis_index("s")
    wid = _worker_id()

    pltpu.sync_copy(denp_hbm.at[0, pl.ds(s * ROWS_S, ROWS_S)], d0_v)
    pltpu.sync_copy(denp_hbm.at[1, pl.ds(s * ROWS_S, ROWS_S)], d1_v)
    for q in range(ROWS_S // 16):
        dt_v[pl.ds(q * 16, 16)] = (d0_v[pl.ds(q * 16, 16)]
                                   + d1_v[pl.ds(q * 16, 16)])
    pltpu.sync_copy(dt_v, den_sh.at[pl.ds(s * ROWS_S, ROWS_S)])
    pltpu.sync_copy(zeros2_hbm.at[pl.ds(s * ROWS_S, ROWS_S)],
                    out_sh.at[pl.ds(s * ROWS_S, ROWS_S)])
    plsc.subcore_barrier()

    def chunk_body(j, carry):
        base = (j * NW + wid) * C
        pltpu.sync_copy(src_hbm.at[pl.ds(base, C)], src_v)
        pltpu.sync_copy(dst_hbm.at[pl.ds(base, C)], dst_v)
        pltpu.sync_copy(ex_hbm.at[pl.ds(base, C)], ex_v)
        cp1 = pltpu.async_copy(hl_hbm.at[src_v], rl_v, sem1)
        cp2 = pltpu.async_copy(den_sh.at[dst_v], den_v, sem2)
        cp1.wait()
        cp2.wait()
        for g in range(C // 16):
            e16 = ex_v[pl.ds(g * 16, 16)]
            d16 = den_v[pl.ds(g * 16, 16)]
            al_v[pl.ds(g * 16, 16)] = e16 / (d16 + 1e-16)

        def edge_body(i, carry2):
            av = plsc.load_gather(al_v, [jnp.full((16,), i, dtype=jnp.int32)])
            for q in range(H // 16):
                rl_v[i, pl.ds(q * 16, 16)] = rl_v[i, pl.ds(q * 16, 16)] * av
            return carry2

        lax.fori_loop(0, C, edge_body, 0, unroll=4)
        pltpu.sync_copy(rl_v, out_sh.at[dst_v], add=True)
        return carry

    lax.fori_loop(0, CHUNKS_W, chunk_body, 0)
    plsc.subcore_barrier()
    pltpu.sync_copy(out_sh.at[pl.ds(s * ROWS_S, ROWS_S)],
                    out_hbm.at[c, pl.ds(s * ROWS_S, ROWS_S)])


def _tc_lin_first(x, Wl, bl, Wr, br):
    def body(x_ref, wl_ref, bl_ref, wr_ref, br_ref, hl_ref, hr_ref):
        a = x_ref[...]
        hl_ref[...] = lax.dot_general(
            a, wl_ref[...], (((1,), (1,)), ((), ())),
            precision=lax.Precision.HIGHEST,
            preferred_element_type=jnp.float32) + bl_ref[...]
        hr_ref[...] = lax.dot_general(
            a, wr_ref[...], (((1,), (1,)), ((), ())),
            precision=lax.Precision.HIGHEST,
            preferred_element_type=jnp.float32) + br_ref[...]

    return pl.pallas_call(
        body,
        out_shape=(jax.ShapeDtypeStruct((N, H), jnp.float32),
                   jax.ShapeDtypeStruct((N, H), jnp.float32)),
    )(x, Wl, bl, Wr, br)


def _tc_lin_next(parts, bprev, Wl, bl, Wr, br):
    def body(p_ref, bp_ref, wl_ref, bl_ref, wr_ref, br_ref, hl_ref, hr_ref):
        a = p_ref[0, :N, :] + p_ref[1, :N, :] + bp_ref[...]
        a = jnp.maximum(a, 0.0)
        hl_ref[...] = lax.dot_general(
            a, wl_ref[...], (((1,), (1,)), ((), ())),
            precision=lax.Precision.HIGHEST,
            preferred_element_type=jnp.float32) + bl_ref[...]
        hr_ref[...] = lax.dot_general(
            a, wr_ref[...], (((1,), (1,)), ((), ())),
            precision=lax.Precision.HIGHEST,
            preferred_element_type=jnp.float32) + br_ref[...]

    return pl.pallas_call(
        body,
        out_shape=(jax.ShapeDtypeStruct((N, H), jnp.float32),
                   jax.ShapeDtypeStruct((N, H), jnp.float32)),
    )(parts, bprev, Wl, bl, Wr, br)


def _tc_pool(parts, bprev, batch2d, Wlin, blin):
    def body(p_ref, bp_ref, bt_ref, wlin_ref, blin_ref, o_ref):
        h = p_ref[0, :N, :] + p_ref[1, :N, :] + bp_ref[...]
        bt = bt_ref[...]
        onehot = (bt == lax.broadcasted_iota(jnp.int32, (N, G), 1))
        onehot = onehot.astype(jnp.float32)
        sums = lax.dot_general(onehot, h, (((0,), (0,)), ((), ())),
                               precision=lax.Precision.HIGHEST,
                               preferred_element_type=jnp.float32)
        ones = jnp.ones((N, 1), jnp.float32)
        cnt = lax.dot_general(onehot, ones, (((0,), (0,)), ((), ())),
                              precision=lax.Precision.HIGHEST,
                              preferred_element_type=jnp.float32)
        hG = sums / jnp.maximum(cnt, 1.0)
        o_ref[...] = lax.dot_general(hG, wlin_ref[...],
                                     (((1,), (1,)), ((), ())),
                                     precision=lax.Precision.HIGHEST,
                                     preferred_element_type=jnp.float32
                                     ) + blin_ref[...]

    return pl.pallas_call(
        body,
        out_shape=jax.ShapeDtypeStruct((G, OUT), jnp.float32),
    )(parts, bprev, batch2d, Wlin, blin)


def kernel(x, edge_index, edge_attr, batch,
           Wl1, bl1, Wr1, br1, We1, att1, b1,
           Wl2, bl2, Wr2, br2, We2, att2, b2,
           Wl3, bl3, Wr3, br3, We3, att3, b3,
           Wlin, blin):
    pad = E_PAD - E
    src = jnp.concatenate(
        [edge_index[0].astype(jnp.int32), jnp.zeros((pad,), jnp.int32)])
    dst = jnp.concatenate(
        [edge_index[1].astype(jnp.int32), jnp.zeros((pad,), jnp.int32)])
    ea = jnp.concatenate(
        [edge_attr[:, 0].astype(jnp.float32), jnp.zeros((pad,), jnp.float32)])
    zeros1 = jnp.zeros((N_PAD,), jnp.float32)
    zeros2 = jnp.zeros((N_PAD, H), jnp.float32)
    batch2d = batch.astype(jnp.int32).reshape(N, 1)

    layers = [
        (Wl1, bl1, Wr1, br1, We1, att1, b1),
        (Wl2, bl2, Wr2, br2, We2, att2, b2),
        (Wl3, bl3, Wr3, br3, We3, att3, b3),
    ]

    parts = None
    bprev = None
    for li, (Wl, bl, Wr, br, We, att, b) in enumerate(layers):
        if li == 0:
            hl, hr = _tc_lin_first(x, Wl, bl.reshape(1, H),
                                   Wr, br.reshape(1, H))
        else:
            hl, hr = _tc_lin_next(parts, bprev.reshape(1, H),
                                  Wl, bl.reshape(1, H), Wr, br.reshape(1, H))
        we_vec = We[:, 0]
        ex, denp = _sc_pass_a(hl, hr, src, dst, ea, we_vec, att, zeros1)
        parts = _sc_pass_b(hl, src, dst, ex, denp, zeros2)
        bprev = b

    return _tc_pool(parts, bprev.reshape(1, H), batch2d, Wlin, blin)


# speedup vs baseline: 3.0193x; 3.0193x over previous
import functools

import jax
import jax.numpy as jnp
from jax import lax
from jax.experimental import pallas as pl
from jax.experimental.pallas import tpu as pltpu
from jax.experimental.pallas import tpu_sc as plsc

N = 10000
E = 320000
H = 128
OUT = 64
G = 64

NC = 2
NS = 16
NW = NC * NS
C = 128
CHUNKS_W = -(-E // (C * NW))
E_PAD = CHUNKS_W * C * NW
N_PAD = 10240
ROWS_S = N_PAD // NS

_mesh = plsc.VectorSubcoreMesh(core_axis_name="c", subcore_axis_name="s")


def _worker_id():
    return lax.axis_index("s") * NC + lax.axis_index("c")


@functools.partial(
    pl.kernel,
    mesh=_mesh,
    compiler_params=pltpu.CompilerParams(needs_layout_passes=False),
    out_type=(
        jax.ShapeDtypeStruct((E_PAD,), jnp.float32),
        jax.ShapeDtypeStruct((NC, N_PAD), jnp.float32),
    ),
    scratch_types=[
        pltpu.VMEM((C,), jnp.int32),
        pltpu.VMEM((C,), jnp.int32),
        pltpu.VMEM((C,), jnp.float32),
        pltpu.VMEM((C, H), jnp.float32),
        pltpu.VMEM((C, H), jnp.float32),
        pltpu.VMEM((C,), jnp.float32),
        pltpu.VMEM((H,), jnp.float32),
        pltpu.VMEM((H,), jnp.float32),
        pltpu.VMEM_SHARED((N_PAD,), jnp.float32),
        pltpu.SemaphoreType.DMA,
        pltpu.SemaphoreType.DMA,
    ],
)
def _sc_pass_a(hl_hbm, hr_hbm, src_hbm, dst_hbm, ea_hbm, we_hbm, att_hbm,
               zeros1_hbm, ex_hbm, den_hbm,
               src_v, dst_v, ea_v, rl_v, rr_v, ex_v, we_v, att_v,
               den_sh, sem1, sem2):
    c = lax.axis_index("c")
    s = lax.axis_index("s")
    wid = _worker_id()

    pltpu.sync_copy(we_hbm, we_v)
    pltpu.sync_copy(att_hbm, att_v)
    pltpu.sync_copy(zeros1_hbm.at[pl.ds(s * ROWS_S, ROWS_S)],
                    den_sh.at[pl.ds(s * ROWS_S, ROWS_S)])
    plsc.subcore_barrier()

    def chunk_body(j, carry):
        base = (j * NW + wid) * C
        pltpu.sync_copy(src_hbm.at[pl.ds(base, C)], src_v)
        pltpu.sync_copy(dst_hbm.at[pl.ds(base, C)], dst_v)
        pltpu.sync_copy(ea_hbm.at[pl.ds(base, C)], ea_v)
        cp1 = pltpu.async_copy(hl_hbm.at[src_v], rl_v, sem1)
        cp2 = pltpu.async_copy(hr_hbm.at[dst_v], rr_v, sem2)
        cp1.wait()
        cp2.wait()
        lanes = lax.iota(jnp.int32, 16)
        for g in range(C // 16):
            rowi = lanes + (g * 16)
            ea16 = ea_v[pl.ds(g * 16, 16)]
            eid16 = lanes + (base + g * 16)

            def f_body(f, acc):
                fv = jnp.full((16,), f, dtype=jnp.int32)
                lv = plsc.load_gather(rl_v, [rowi, fv])
                rv = plsc.load_gather(rr_v, [rowi, fv])
                wf = plsc.load_gather(we_v, [fv])
                af = plsc.load_gather(att_v, [fv])
                m = lv + rv + ea16 * wf
                m = jnp.maximum(m, 0.2 * m)
                return acc + m * af

            acc = lax.fori_loop(0, H, f_body, jnp.zeros((16,), jnp.float32),
                                unroll=8)
            exv = jnp.exp(acc)
            exv = jnp.where(eid16 < E, exv, 0.0)
            ex_v[pl.ds(g * 16, 16)] = exv
        pltpu.sync_copy(ex_v, ex_hbm.at[pl.ds(base, C)])
        pltpu.sync_copy(ex_v, den_sh.at[dst_v], add=True)
        return carry

    lax.fori_loop(0, CHUNKS_W, chunk_body, 0)
    plsc.subcore_barrier()
    pltpu.sync_copy(den_sh.at[pl.ds(s * ROWS_S, ROWS_S)],
                    den_hbm.at[c, pl.ds(s * ROWS_S, ROWS_S)])


@functools.partial(
    pl.kernel,
    mesh=_mesh,
    compiler_params=pltpu.CompilerParams(needs_layout_passes=False),
    out_type=jax.ShapeDtypeStruct((NC, N_PAD, H), jnp.float32),
    scratch_types=[
        pltpu.VMEM((C,), jnp.int32),
        pltpu.VMEM((C,), jnp.int32),
        pltpu.VMEM((C,), jnp.float32),
        pltpu.VMEM((C,), jnp.float32),
        pltpu.VMEM((C,), jnp.float32),
        pltpu.VMEM((C, H), jnp.float32),
        pltpu.VMEM((ROWS_S,), jnp.float32),
        pltpu.VMEM((ROWS_S,), jnp.float32),
        pltpu.VMEM((ROWS_S,), jnp.float32),
        pltpu.VMEM_SHARED((N_PAD,), jnp.float32),
        pltpu.VMEM_SHARED((N_PAD, H), jnp.float32),
        pltpu.SemaphoreType.DMA,
        pltpu.SemaphoreType.DMA,
    ],
)
def _sc_pass_b(hl_hbm, src_hbm, dst_hbm, ex_hbm, denp_hbm, zeros2_hbm,
               out_hbm,
               src_v, dst_v, ex_v, den_v, al_v, rl_v, d0_v, d1_v, dt_v,
               den_sh, out_sh, sem1, sem2):
    c = lax.axis_index("c")
    s = lax.axis_index("s")
    wid = _worker_id()

    pltpu.sync_copy(denp_hbm.at[0, pl.ds(s * ROWS_S, ROWS_S)], d0_v)
    pltpu.sync_copy(denp_hbm.at[1, pl.ds(s * ROWS_S, ROWS_S)], d1_v)
    for q in range(ROWS_S // 16):
        dt_v[pl.ds(q * 16, 16)] = (d0_v[pl.ds(q * 16, 16)]
                                   + d1_v[pl.ds(q * 16, 16)])
    pltpu.sync_copy(dt_v, den_sh.at[pl.ds(s * ROWS_S, ROWS_S)])
    pltpu.sync_copy(zeros2_hbm.at[pl.ds(s * ROWS_S, ROWS_S)],
                    out_sh.at[pl.ds(s * ROWS_S, ROWS_S)])
    plsc.subcore_barrier()

    def chunk_body(j, carry):
        base = (j * NW + wid) * C
        pltpu.sync_copy(src_hbm.at[pl.ds(base, C)], src_v)
        pltpu.sync_copy(dst_hbm.at[pl.ds(base, C)], dst_v)
        pltpu.sync_copy(ex_hbm.at[pl.ds(base, C)], ex_v)
        cp1 = pltpu.async_copy(hl_hbm.at[src_v], rl_v, sem1)
        cp2 = pltpu.async_copy(den_sh.at[dst_v], den_v, sem2)
        cp1.wait()
        cp2.wait()
        for g in range(C // 16):
            e16 = ex_v[pl.ds(g * 16, 16)]
            d16 = den_v[pl.ds(g * 16, 16)]
            al_v[pl.ds(g * 16, 16)] = e16 / (d16 + 1e-16)

        def edge_body(i, carry2):
            av = plsc.load_gather(al_v, [jnp.full((16,), i, dtype=jnp.int32)])
            for q in range(H // 16):
                rl_v[i, pl.ds(q * 16, 16)] = rl_v[i, pl.ds(q * 16, 16)] * av
            return carry2

        lax.fori_loop(0, C, edge_body, 0, unroll=4)
        pltpu.sync_copy(rl_v, out_sh.at[dst_v], add=True)
        return carry

    lax.fori_loop(0, CHUNKS_W, chunk_body, 0)
    plsc.subcore_barrier()
    pltpu.sync_copy(out_sh.at[pl.ds(s * ROWS_S, ROWS_S)],
                    out_hbm.at[c, pl.ds(s * ROWS_S, ROWS_S)])


def _tc_lin_first(x, Wl, bl, Wr, br):
    def body(x_ref, wl_ref, bl_ref, wr_ref, br_ref, hl_ref, hr_ref):
        a = x_ref[...]
        hl_ref[...] = lax.dot_general(
            a, wl_ref[...], (((1,), (1,)), ((), ())),
            precision=lax.Precision.HIGHEST,
            preferred_element_type=jnp.float32) + bl_ref[...]
        hr_ref[...] = lax.dot_general(
            a, wr_ref[...], (((1,), (1,)), ((), ())),
            precision=lax.Precision.HIGHEST,
            preferred_element_type=jnp.float32) + br_ref[...]

    return pl.pallas_call(
        body,
        out_shape=(jax.ShapeDtypeStruct((N, H), jnp.float32),
                   jax.ShapeDtypeStruct((N, H), jnp.float32)),
    )(x, Wl, bl, Wr, br)


def _tc_lin_next(parts, bprev, Wl, bl, Wr, br):
    def body(p_ref, bp_ref, wl_ref, bl_ref, wr_ref, br_ref, hl_ref, hr_ref):
        a = p_ref[0, :N, :] + p_ref[1, :N, :] + bp_ref[...]
        a = jnp.maximum(a, 0.0)
        hl_ref[...] = lax.dot_general(
            a, wl_ref[...], (((1,), (1,)), ((), ())),
            precision=lax.Precision.HIGHEST,
            preferred_element_type=jnp.float32) + bl_ref[...]
        hr_ref[...] = lax.dot_general(
            a, wr_ref[...], (((1,), (1,)), ((), ())),
            precision=lax.Precision.HIGHEST,
            preferred_element_type=jnp.float32) + br_ref[...]

    return pl.pallas_call(
        body,
        out_shape=(jax.ShapeDtypeStruct((N, H), jnp.float32),
                   jax.ShapeDtypeStruct((N, H), jnp.float32)),
    )(parts, bprev, Wl, bl, Wr, br)


def _tc_pool(parts, bprev, batch2d, Wlin, blin):
    def body(p_ref, bp_ref, bt_ref, wlin_ref, blin_ref, o_ref):
        h = p_ref[0, :N, :] + p_ref[1, :N, :] + bp_ref[...]
        bt = bt_ref[...]
        onehot = (bt == lax.broadcasted_iota(jnp.int32, (N, G), 1))
        onehot = onehot.astype(jnp.float32)
        sums = lax.dot_general(onehot, h, (((0,), (0,)), ((), ())),
                               precision=lax.Precision.HIGHEST,
                               preferred_element_type=jnp.float32)
        ones = jnp.ones((N, 1), jnp.float32)
        cnt = lax.dot_general(onehot, ones, (((0,), (0,)), ((), ())),
                              precision=lax.Precision.HIGHEST,
                              preferred_element_type=jnp.float32)
        hG = sums / jnp.maximum(cnt, 1.0)
        o_ref[...] = lax.dot_general(hG, wlin_ref[...],
                                     (((1,), (1,)), ((), ())),
                                     precision=lax.Precision.HIGHEST,
                                     preferred_element_type=jnp.float32
                                     ) + blin_ref[...]

    return pl.pallas_call(
        body,
        out_shape=jax.ShapeDtypeStruct((G, OUT), jnp.float32),
    )(parts, bprev, batch2d, Wlin, blin)


def kernel(x, edge_index, edge_attr, batch,
           Wl1, bl1, Wr1, br1, We1, att1, b1,
           Wl2, bl2, Wr2, br2, We2, att2, b2,
           Wl3, bl3, Wr3, br3, We3, att3, b3,
           Wlin, blin):
    pad = E_PAD - E
    src = jnp.concatenate(
        [edge_index[0].astype(jnp.int32), jnp.zeros((pad,), jnp.int32)])
    dst = jnp.concatenate(
        [edge_index[1].astype(jnp.int32), jnp.zeros((pad,), jnp.int32)])
    ea = jnp.concatenate(
        [edge_attr[:, 0].astype(jnp.float32), jnp.zeros((pad,), jnp.float32)])
    zeros1 = jnp.zeros((N_PAD,), jnp.float32)
    zeros2 = jnp.zeros((N_PAD, H), jnp.float32)
    batch2d = batch.astype(jnp.int32).reshape(N, 1)

    layers = [
        (Wl1, bl1, Wr1, br1, We1, att1, b1),
        (Wl2, bl2, Wr2, br2, We2, att2, b2),
        (Wl3, bl3, Wr3, br3, We3, att3, b3),
    ]

    parts = None
    bprev = None
    for li, (Wl, bl, Wr, br, We, att, b) in enumerate(layers):
        if li == 0:
            hl, hr = _tc_lin_first(x, Wl, bl.reshape(1, H),
                                   Wr, br.reshape(1, H))
        else:
            hl, hr = _tc_lin_next(parts, bprev.reshape(1, H),
                                  Wl, bl.reshape(1, H), Wr, br.reshape(1, H))
        we_vec = We[:, 0]
        ex, denp = _sc_pass_a(hl, hr, src, dst, ea, we_vec, att, zeros1)
        parts = _sc_pass_b(hl, src, dst, ex, denp, zeros2)
        bprev = b

    return _tc_pool(parts, bprev.reshape(1, H), batch2d, Wlin, blin)
